# Initial kernel scaffold; baseline (speedup 1.0000x reference)
import jax
import jax.numpy as jnp
from jax.experimental import pallas as pl


def kernel(fleet_plan, fleet_plan_mask, token_table, pos_table, fc_w, fc_b):
    raise NotImplementedError("write your pallas kernel here")


# speedup vs baseline: 8.5543x; 8.5543x over previous
import jax
import jax.numpy as jnp
from jax.experimental import pallas as pl
from jax.experimental.pallas import tpu as pltpu

_BS, _NR, _PLAN = 1024, 26, 20
_VOCAB, _POS, _DIM = 18, 20, 128
_N = _BS * _NR
_ROWS = 1024
_GRID = _N // _ROWS


def _body(fp_ref, keep_ref, tt_ref, pt_ref, wt_ref, b_ref, out_ref):
    fp = fp_ref[...]
    keep = keep_ref[...]

    tok_w = jnp.dot(tt_ref[...], wt_ref[...], preferred_element_type=jnp.float32)
    pos_wb = jnp.dot(pt_ref[...], wt_ref[...], preferred_element_type=jnp.float32) + b_ref[...]

    iota_v = jax.lax.broadcasted_iota(jnp.int32, (_ROWS, _VOCAB), 1)
    counts = jnp.zeros((_ROWS, _VOCAB), jnp.float32)
    for t in range(_PLAN):
        onehot = (fp[:, t][:, None] == iota_v).astype(jnp.float32)
        counts = counts + keep[:, t][:, None] * onehot

    out_ref[...] = (
        jnp.dot(counts, tok_w, preferred_element_type=jnp.float32)
        + jnp.dot(keep, pos_wb, preferred_element_type=jnp.float32)
    )


def kernel(fleet_plan, fleet_plan_mask, token_table, pos_table, fc_w, fc_b):
    fp2 = fleet_plan.reshape(_N, _PLAN).astype(jnp.int32)
    keep2 = (1.0 - fleet_plan_mask.reshape(_N, _PLAN).astype(jnp.float32))

    out = pl.pallas_call(
        _body,
        grid=(_GRID,),
        in_specs=[
            pl.BlockSpec((_ROWS, _PLAN), lambda i: (i, 0)),
            pl.BlockSpec((_ROWS, _PLAN), lambda i: (i, 0)),
            pl.BlockSpec((_VOCAB, _DIM), lambda i: (0, 0)),
            pl.BlockSpec((_POS, _DIM), lambda i: (0, 0)),
            pl.BlockSpec((_DIM, _DIM), lambda i: (0, 0)),
            pl.BlockSpec((1, _DIM), lambda i: (0, 0)),
        ],
        out_specs=pl.BlockSpec((_ROWS, _DIM), lambda i: (i, 0)),
        out_shape=jax.ShapeDtypeStruct((_N, _DIM), jnp.float32),
    )(fp2, keep2, token_table, pos_table, fc_w.T, fc_b.reshape(1, _DIM))

    return out.reshape(_BS, _NR, _DIM)


# speedup vs baseline: 21.8317x; 2.5521x over previous
import jax
import jax.numpy as jnp
from jax.experimental import pallas as pl
from jax.experimental.pallas import tpu as pltpu

_BS, _NR, _PLAN = 1024, 26, 20
_VOCAB, _POS, _DIM = 18, 20, 128
_N = _BS * _NR
_COLS = 2048
_GRID = _N // _COLS

_DN0 = (((0,), (0,)), ((), ()))


def _body(fpt_ref, keept_ref, tt_ref, pt_ref, wt_ref, b_ref, out_ref):
    fpt = fpt_ref[...]
    keept = keept_ref[...]

    tok_w = jnp.dot(tt_ref[...], wt_ref[...], preferred_element_type=jnp.float32)
    pos_wb = jnp.dot(pt_ref[...], wt_ref[...], preferred_element_type=jnp.float32) + b_ref[...]

    cols = [
        jnp.sum(jnp.where(fpt == v, keept, 0.0), axis=0, keepdims=True)
        for v in range(_VOCAB)
    ]
    counts_t = jnp.concatenate(cols, axis=0)

    out_ref[...] = (
        jax.lax.dot_general(counts_t, tok_w, _DN0, preferred_element_type=jnp.float32)
        + jax.lax.dot_general(keept, pos_wb, _DN0, preferred_element_type=jnp.float32)
    )


def kernel(fleet_plan, fleet_plan_mask, token_table, pos_table, fc_w, fc_b):
    fpt = fleet_plan.reshape(_N, _PLAN).astype(jnp.int32).T
    keept = (1.0 - fleet_plan_mask.reshape(_N, _PLAN).astype(jnp.float32)).T

    out = pl.pallas_call(
        _body,
        grid=(_GRID,),
        in_specs=[
            pl.BlockSpec((_PLAN, _COLS), lambda i: (0, i)),
            pl.BlockSpec((_PLAN, _COLS), lambda i: (0, i)),
            pl.BlockSpec((_VOCAB, _DIM), lambda i: (0, 0)),
            pl.BlockSpec((_POS, _DIM), lambda i: (0, 0)),
            pl.BlockSpec((_DIM, _DIM), lambda i: (0, 0)),
            pl.BlockSpec((1, _DIM), lambda i: (0, 0)),
        ],
        out_specs=pl.BlockSpec((_COLS, _DIM), lambda i: (i, 0)),
        out_shape=jax.ShapeDtypeStruct((_N, _DIM), jnp.float32),
    )(fpt, keept, token_table, pos_table, fc_w.T, fc_b.reshape(1, _DIM))

    return out.reshape(_BS, _NR, _DIM)


# speedup vs baseline: 28.4731x; 1.3042x over previous
import jax
import jax.numpy as jnp
from jax.experimental import pallas as pl
from jax.experimental.pallas import tpu as pltpu

_BS, _NR, _PLAN = 1024, 26, 20
_VOCAB, _POS, _DIM = 18, 20, 128
_BB = 128
_GRID = _BS // _BB
_C = _BB * _NR

_DN0 = (((0,), (0,)), ((), ()))


def _body(fp_ref, mask_ref, tt_ref, pt_ref, wt_ref, b_ref, out_ref):
    fp = fp_ref[...].reshape(_C, _PLAN)
    keep = 1.0 - mask_ref[...].reshape(_C, _PLAN).astype(jnp.float32)

    fpt = fp.T
    keept = keep.T

    tok_w = jnp.dot(tt_ref[...], wt_ref[...], preferred_element_type=jnp.float32)
    pos_wb = jnp.dot(pt_ref[...], wt_ref[...], preferred_element_type=jnp.float32) + b_ref[...]

    cols = [
        jnp.sum(jnp.where(fpt == v, keept, 0.0), axis=0, keepdims=True)
        for v in range(_VOCAB)
    ]
    counts_t = jnp.concatenate(cols, axis=0)

    out = (
        jax.lax.dot_general(counts_t, tok_w, _DN0, preferred_element_type=jnp.float32)
        + jax.lax.dot_general(keept, pos_wb, _DN0, preferred_element_type=jnp.float32)
    )
    out_ref[...] = out.reshape(_BB, _NR, _DIM)


def kernel(fleet_plan, fleet_plan_mask, token_table, pos_table, fc_w, fc_b):
    out = pl.pallas_call(
        _body,
        grid=(_GRID,),
        in_specs=[
            pl.BlockSpec((_BB, _NR, _PLAN), lambda i: (i, 0, 0)),
            pl.BlockSpec((_BB, _NR, _PLAN), lambda i: (i, 0, 0)),
            pl.BlockSpec((_VOCAB, _DIM), lambda i: (0, 0)),
            pl.BlockSpec((_POS, _DIM), lambda i: (0, 0)),
            pl.BlockSpec((_DIM, _DIM), lambda i: (0, 0)),
            pl.BlockSpec((1, _DIM), lambda i: (0, 0)),
        ],
        out_specs=pl.BlockSpec((_BB, _NR, _DIM), lambda i: (i, 0, 0)),
        out_shape=jax.ShapeDtypeStruct((_BS, _NR, _DIM), jnp.float32),
    )(fleet_plan.astype(jnp.int32), fleet_plan_mask, token_table, pos_table,
      fc_w.T, fc_b.reshape(1, _DIM))

    return out


# speedup vs baseline: 29.6755x; 1.0422x over previous
import jax
import jax.numpy as jnp
from jax.experimental import pallas as pl
from jax.experimental.pallas import tpu as pltpu

_BS, _NR, _PLAN = 1024, 26, 20
_VOCAB, _POS, _DIM = 18, 20, 128
_N = _BS * _NR
_BB = 128
_GRID = _BS // _BB
_C = _BB * _NR

_DN0 = (((0,), (0,)), ((), ()))


def _body(fpt_ref, keept_ref, tt_ref, pt_ref, wt_ref, b_ref, out_ref):
    fpt = fpt_ref[...]
    keept = keept_ref[...]

    tok_w = jnp.dot(tt_ref[...], wt_ref[...], preferred_element_type=jnp.float32)
    pos_wb = jnp.dot(pt_ref[...], wt_ref[...], preferred_element_type=jnp.float32) + b_ref[...]

    cols = [
        jnp.sum(jnp.where(fpt == v, keept, 0.0), axis=0, keepdims=True)
        for v in range(_VOCAB)
    ]
    counts_t = jnp.concatenate(cols, axis=0)

    out = (
        jax.lax.dot_general(counts_t, tok_w, _DN0, preferred_element_type=jnp.float32)
        + jax.lax.dot_general(keept, pos_wb, _DN0, preferred_element_type=jnp.float32)
    )
    out_ref[...] = out.reshape(_BB, _NR, _DIM)


def kernel(fleet_plan, fleet_plan_mask, token_table, pos_table, fc_w, fc_b):
    fpt = fleet_plan.reshape(_N, _PLAN).astype(jnp.int32).T
    keept = (1.0 - fleet_plan_mask.reshape(_N, _PLAN).astype(jnp.float32)).T

    out = pl.pallas_call(
        _body,
        grid=(_GRID,),
        in_specs=[
            pl.BlockSpec((_PLAN, _C), lambda i: (0, i)),
            pl.BlockSpec((_PLAN, _C), lambda i: (0, i)),
            pl.BlockSpec((_VOCAB, _DIM), lambda i: (0, 0)),
            pl.BlockSpec((_POS, _DIM), lambda i: (0, 0)),
            pl.BlockSpec((_DIM, _DIM), lambda i: (0, 0)),
            pl.BlockSpec((1, _DIM), lambda i: (0, 0)),
        ],
        out_specs=pl.BlockSpec((_BB, _NR, _DIM), lambda i: (i, 0, 0)),
        out_shape=jax.ShapeDtypeStruct((_BS, _NR, _DIM), jnp.float32),
    )(fpt, keept, token_table, pos_table, fc_w.T, fc_b.reshape(1, _DIM))

    return out
